# baseline (speedup 1.0000x reference)
import jax
import jax.numpy as jnp
from jax.experimental import pallas as pl
from jax.experimental.pallas import tpu as pltpu

_CHUNKS = 12
_NBUF = 12


def _dma_pipeline_body(v_ref, o_ref, buf, in_sems, out_sems):
    def in_copy(k):
        s = k % _NBUF
        return pltpu.make_async_copy(v_ref.at[k], buf.at[s], in_sems.at[s])

    def out_copy(k):
        s = k % _NBUF
        return pltpu.make_async_copy(buf.at[s], o_ref.at[k], out_sems.at[s])

    for k in range(_NBUF):
        in_copy(k).start()
    for k in range(_CHUNKS):
        in_copy(k).wait()
        out_copy(k).start()
        nxt = k + _NBUF
        if nxt < _CHUNKS:
            out_copy(k).wait()
            in_copy(nxt).start()
    for k in range(_CHUNKS - _NBUF, _CHUNKS):
        out_copy(k).wait()


def kernel(queries, keys, values):
    b, l, h, d = values.shape
    vt = jnp.transpose(values, (0, 2, 3, 1)).reshape(_CHUNKS, (b * h * d) // _CHUNKS, l)
    out = pl.pallas_call(
        _dma_pipeline_body,
        in_specs=[pl.BlockSpec(memory_space=pltpu.MemorySpace.HBM)],
        out_specs=pl.BlockSpec(memory_space=pltpu.MemorySpace.HBM),
        out_shape=jax.ShapeDtypeStruct(vt.shape, vt.dtype),
        scratch_shapes=[
            pltpu.VMEM((_NBUF,) + vt.shape[1:], vt.dtype),
            pltpu.SemaphoreType.DMA((_NBUF,)),
            pltpu.SemaphoreType.DMA((_NBUF,)),
        ],
    )(vt)
    return jnp.transpose(out.reshape(b, h, d, l), (0, 1, 3, 2))


# speedup vs baseline: 1.0082x; 1.0082x over previous
import jax
import jax.numpy as jnp
from jax.experimental import pallas as pl
from jax.experimental.pallas import tpu as pltpu

_CHUNKS = 8
_NBUF = 8


def _dma_pipeline_body(v_ref, o_ref, buf, in_sems, out_sems):
    def in_copy(k):
        s = k % _NBUF
        return pltpu.make_async_copy(v_ref.at[k], buf.at[s], in_sems.at[s])

    def out_copy(k):
        s = k % _NBUF
        return pltpu.make_async_copy(buf.at[s], o_ref.at[k], out_sems.at[s])

    for k in range(_NBUF):
        in_copy(k).start()
    for k in range(_CHUNKS):
        in_copy(k).wait()
        out_copy(k).start()
        nxt = k + _NBUF
        if nxt < _CHUNKS:
            out_copy(k).wait()
            in_copy(nxt).start()
    for k in range(_CHUNKS - _NBUF, _CHUNKS):
        out_copy(k).wait()


def kernel(queries, keys, values):
    b, l, h, d = values.shape
    vt = jnp.transpose(values, (0, 2, 3, 1)).reshape(_CHUNKS, (b * h * d) // _CHUNKS, l)
    out = pl.pallas_call(
        _dma_pipeline_body,
        in_specs=[pl.BlockSpec(memory_space=pltpu.MemorySpace.HBM)],
        out_specs=pl.BlockSpec(memory_space=pltpu.MemorySpace.HBM),
        out_shape=jax.ShapeDtypeStruct(vt.shape, vt.dtype),
        scratch_shapes=[
            pltpu.VMEM((_NBUF,) + vt.shape[1:], vt.dtype),
            pltpu.SemaphoreType.DMA((_NBUF,)),
            pltpu.SemaphoreType.DMA((_NBUF,)),
        ],
    )(vt)
    return jnp.transpose(out.reshape(b, h, d, l), (0, 1, 3, 2))
